# Initial kernel scaffold; baseline (speedup 1.0000x reference)
import jax
import jax.numpy as jnp
from jax.experimental import pallas as pl


def kernel(x, variable_seq, W_var):
    raise NotImplementedError("write your pallas kernel here")


# speedup vs baseline: 2.2186x; 2.2186x over previous
import functools

import jax
import jax.numpy as jnp
from jax import lax
from jax.experimental import pallas as pl
from jax.experimental.pallas import tpu as pltpu
from jax.experimental.pallas import tpu_sc as plsc

EMBED = 64
LANES = 16
NC = 2
NS = 16
NW = NC * NS
C = 512
SUB = 128


def _make_embed_add(n_rows: int):
    assert n_rows % (NW * C) == 0
    n_w = n_rows // NW
    num_chunks = n_w // C
    mesh = plsc.VectorSubcoreMesh(core_axis_name="c", subcore_axis_name="s")

    @functools.partial(
        pl.kernel,
        out_type=jax.ShapeDtypeStruct((n_rows, EMBED), jnp.float32),
        mesh=mesh,
        scratch_types=[
            pltpu.VMEM((C,), jnp.int32),
            pltpu.VMEM((C, EMBED), jnp.float32),
            pltpu.VMEM((C, EMBED), jnp.float32),
            pltpu.SemaphoreType.DMA,
            pltpu.SemaphoreType.DMA,
        ],
        compiler_params=pltpu.CompilerParams(use_tc_tiling_on_sc=False),
    )
    def embed_add(x_hbm, idx_hbm, table_hbm, out_hbm, idx_v, rows_v, x_v,
                  sem_g, sem_x):
        wid = lax.axis_index("s") * NC + lax.axis_index("c")
        base = wid * n_w

        def chunk_body(g, carry):
            off = base + g * C
            pltpu.sync_copy(idx_hbm.at[pl.ds(off, C)], idx_v)
            gathers = []
            for j in range(C // SUB):
                gathers.append(pltpu.async_copy(
                    table_hbm.at[idx_v.at[pl.ds(j * SUB, SUB)]],
                    rows_v.at[pl.ds(j * SUB, SUB)],
                    sem_g,
                ))
            cx = pltpu.async_copy(x_hbm.at[pl.ds(off, C)], x_v, sem_x)
            for cg in gathers:
                cg.wait()
            cx.wait()

            def add_body(i, carry2):
                for r in range(4):
                    for j in range(EMBED // LANES):
                        sl = pl.ds(j * LANES, LANES)
                        plsc.addupdate(rows_v.at[i * 4 + r, sl],
                                       x_v[i * 4 + r, sl])
                return carry2

            lax.fori_loop(0, C // 4, add_body, 0, unroll=False)
            pltpu.sync_copy(rows_v, out_hbm.at[pl.ds(off, C)])
            return carry

        lax.fori_loop(0, num_chunks, chunk_body, 0, unroll=False)

    return embed_add


def kernel(x, variable_seq, W_var):
    B, L, E = x.shape
    n_rows = B * L
    xf = x.reshape(n_rows, E)
    idx = variable_seq.reshape(n_rows).astype(jnp.int32)
    out = _make_embed_add(n_rows)(xf, idx, W_var)
    return out.reshape(B, L, E)


# speedup vs baseline: 2.4921x; 1.1233x over previous
import functools

import jax
import jax.numpy as jnp
from jax import lax
from jax.experimental import pallas as pl
from jax.experimental.pallas import tpu as pltpu
from jax.experimental.pallas import tpu_sc as plsc

EMBED = 64
LANES = 16
NC = 2
NS = 16
NW = NC * NS
C = 256
SUB = 128


def _make_embed_add(n_rows: int):
    assert n_rows % (NW * C) == 0
    n_w = n_rows // NW
    nch = n_w // C
    assert nch % 2 == 0 and nch >= 4
    mesh = plsc.VectorSubcoreMesh(core_axis_name="c", subcore_axis_name="s")

    @functools.partial(
        pl.kernel,
        out_type=jax.ShapeDtypeStruct((n_rows, EMBED), jnp.float32),
        mesh=mesh,
        scratch_types=[
            pltpu.VMEM((C,), jnp.int32),
            pltpu.VMEM((C,), jnp.int32),
            pltpu.VMEM((C, EMBED), jnp.float32),
            pltpu.VMEM((C, EMBED), jnp.float32),
            pltpu.VMEM((C, EMBED), jnp.float32),
            pltpu.VMEM((C, EMBED), jnp.float32),
            pltpu.SemaphoreType.DMA,
            pltpu.SemaphoreType.DMA,
            pltpu.SemaphoreType.DMA,
            pltpu.SemaphoreType.DMA,
            pltpu.SemaphoreType.DMA,
            pltpu.SemaphoreType.DMA,
            pltpu.SemaphoreType.DMA,
            pltpu.SemaphoreType.DMA,
        ],
        compiler_params=pltpu.CompilerParams(use_tc_tiling_on_sc=False),
    )
    def embed_add(x_hbm, idx_hbm, table_hbm, out_hbm,
                  idx0, idx1, r0, r1, xv0, xv1,
                  si0, si1, sg0, sg1, sx0, sx1, so0, so1):
        idx_v = (idx0, idx1)
        rows_v = (r0, r1)
        x_v = (xv0, xv1)
        sem_i = (si0, si1)
        sem_g = (sg0, sg1)
        sem_x = (sx0, sx1)
        sem_o = (so0, so1)
        wid = lax.axis_index("s") * NC + lax.axis_index("c")
        base = wid * n_w

        def row_slice(g):
            return pl.ds(base + g * C, C)

        def start_idx(g, b):
            pltpu.async_copy(idx_hbm.at[row_slice(g)], idx_v[b], sem_i[b])

        def wait_idx(g, b):
            pltpu.make_async_copy(
                idx_hbm.at[row_slice(g)], idx_v[b], sem_i[b]).wait()

        def start_gx(g, b):
            for j in range(C // SUB):
                ds = pl.ds(j * SUB, SUB)
                pltpu.async_copy(
                    table_hbm.at[idx_v[b].at[ds]], rows_v[b].at[ds], sem_g[b])
            pltpu.async_copy(x_hbm.at[row_slice(g)], x_v[b], sem_x[b])

        def wait_gx(g, b):
            for j in range(C // SUB):
                ds = pl.ds(j * SUB, SUB)
                pltpu.make_async_copy(
                    table_hbm.at[idx_v[b].at[ds]],
                    rows_v[b].at[ds], sem_g[b]).wait()
            pltpu.make_async_copy(
                x_hbm.at[row_slice(g)], x_v[b], sem_x[b]).wait()

        def start_out(g, b):
            pltpu.async_copy(rows_v[b], out_hbm.at[row_slice(g)], sem_o[b])

        def wait_out(g, b):
            pltpu.make_async_copy(
                rows_v[b], out_hbm.at[row_slice(g)], sem_o[b]).wait()

        def add_chunk(b):
            rv, xv = rows_v[b], x_v[b]

            def add_body(i, carry):
                for j in range(EMBED // LANES):
                    sl = pl.ds(j * LANES, LANES)
                    plsc.addupdate(rv.at[i, sl], xv[i, sl])
                return carry

            lax.fori_loop(0, C, add_body, 0, unroll=8)

        start_idx(0, 0)
        start_idx(1, 1)
        wait_idx(0, 0)
        start_gx(0, 0)

        def pair_body(gg, carry):
            for b in range(2):
                g = 2 * gg + b
                o = b ^ 1
                wait_gx(g, b)

                @pl.when(g + 2 < nch)
                def _():
                    start_idx(g + 2, b)

                @pl.when(g + 1 < nch)
                def _():
                    wait_idx(g + 1, o)

                    @pl.when(g >= 1)
                    def _():
                        wait_out(g - 1, o)

                    start_gx(g + 1, o)

                add_chunk(b)
                start_out(g, b)
            return carry

        lax.fori_loop(0, nch // 2, pair_body, 0, unroll=False)
        wait_out(nch - 2, 0)
        wait_out(nch - 1, 1)

    return embed_add


def kernel(x, variable_seq, W_var):
    B, L, E = x.shape
    n_rows = B * L
    xf = x.reshape(n_rows, E)
    idx = variable_seq.reshape(n_rows).astype(jnp.int32)
    out = _make_embed_add(n_rows)(xf, idx, W_var)
    return out.reshape(B, L, E)


# speedup vs baseline: 6.1145x; 2.4535x over previous
import functools

import jax
import jax.numpy as jnp
from jax import lax
from jax.experimental import pallas as pl
from jax.experimental.pallas import tpu as pltpu
from jax.experimental.pallas import tpu_sc as plsc

EMBED = 64
LANES = 16
NC = 2
NS = 16
NW = NC * NS
L_ROWS = 200
BATCH = 4096
VOCAB = 100000


def _make_embed_add():
    mesh = plsc.VectorSubcoreMesh(core_axis_name="c", subcore_axis_name="s")

    @functools.partial(
        pl.kernel,
        out_type=jax.ShapeDtypeStruct((L_ROWS * EMBED, BATCH), jnp.float32),
        mesh=mesh,
        scratch_types=[
            pltpu.VMEM((VOCAB,), jnp.float32),
            pltpu.VMEM((BATCH,), jnp.float32),
            pltpu.VMEM((BATCH,), jnp.float32),
            pltpu.VMEM((BATCH,), jnp.float32),
            pltpu.VMEM((BATCH,), jnp.float32),
            pltpu.VMEM((BATCH,), jnp.int32),
            pltpu.VMEM((BATCH,), jnp.int32),
            pltpu.SemaphoreType.DMA,
            pltpu.SemaphoreType.DMA,
            pltpu.SemaphoreType.DMA,
            pltpu.SemaphoreType.DMA,
            pltpu.SemaphoreType.DMA,
            pltpu.SemaphoreType.DMA,
            pltpu.SemaphoreType.DMA,
            pltpu.SemaphoreType.DMA,
            pltpu.SemaphoreType.DMA,
            pltpu.SemaphoreType.DMA,
        ],
        compiler_params=pltpu.CompilerParams(
            use_tc_tiling_on_sc=True, needs_layout_passes=False),
    )
    def embed_add(x_hbm, idx_hbm, w_hbm, out_hbm,
                  wrow, xo0, xo1, xo2, xo3, id0, id1,
                  si0, si1, sx0, sx1, sx2, sx3, so0, so1, so2, so3):
        xo = (xo0, xo1, xo2, xo3)
        idv = (id0, id1)
        sem_i = (si0, si1)
        sem_x = (sx0, sx1, sx2, sx3)
        sem_o = (so0, so1, so2, so3)
        wid = lax.axis_index("s") * NC + lax.axis_index("c")

        for p in range(2):
            e = wid + NW * p

            def xrow(l):
                return x_hbm.at[l * EMBED + e]

            def orow(l):
                return out_hbm.at[l * EMBED + e]

            def start_idx(l, bi):
                pltpu.async_copy(idx_hbm.at[l], idv[bi], sem_i[bi])

            def wait_idx(l, bi):
                pltpu.make_async_copy(idx_hbm.at[l], idv[bi], sem_i[bi]).wait()

            def start_x(l, bx):
                pltpu.async_copy(xrow(l), xo[bx], sem_x[bx])

            def wait_x(l, bx):
                pltpu.make_async_copy(xrow(l), xo[bx], sem_x[bx]).wait()

            def start_out(l, bx):
                pltpu.async_copy(xo[bx], orow(l), sem_o[bx])

            def wait_out(l, bx):
                pltpu.make_async_copy(xo[bx], orow(l), sem_o[bx]).wait()

            def gather_add(bi, bx):
                iv = idv[bi]
                ov = xo[bx]

                def gbody(k, carry):
                    sl = pl.ds(k * LANES, LANES)
                    vi = iv[sl]
                    g = plsc.load_gather(wrow, [vi])
                    plsc.addupdate(ov.at[sl], g)
                    return carry

                lax.fori_loop(0, BATCH // LANES, gbody, 0, unroll=8)

            pltpu.sync_copy(w_hbm.at[e], wrow)
            start_idx(0, 0)
            start_idx(1, 1)
            start_x(0, 0)
            start_x(1, 1)

            def quad_body(qq, carry):
                for j in range(4):
                    l = 4 * qq + j
                    bi = j % 2
                    wait_idx(l, bi)
                    wait_x(l, j)
                    gather_add(bi, j)

                    @pl.when(l + 2 < L_ROWS)
                    def _():
                        start_idx(l + 2, bi)

                    start_out(l, j)

                    @pl.when(l >= 2)
                    def _():
                        wait_out(l - 2, (j + 2) % 4)

                    @pl.when(l + 2 < L_ROWS)
                    def _():
                        start_x(l + 2, (j + 2) % 4)
                return carry

            lax.fori_loop(0, L_ROWS // 4, quad_body, 0, unroll=False)
            wait_out(L_ROWS - 2, 2)
            wait_out(L_ROWS - 1, 3)

    return embed_add


def kernel(x, variable_seq, W_var):
    B, L, E = x.shape
    x2 = x.transpose(1, 2, 0).reshape(L * E, B)
    idx_t = variable_seq.transpose(1, 0).astype(jnp.int32)
    w_t = W_var.transpose(1, 0)
    out2 = _make_embed_add()(x2, idx_t, w_t)
    return out2.reshape(L, E, B).transpose(2, 0, 1)
